# baseline (speedup 1.0000x reference)
import functools

import jax
import jax.numpy as jnp
from jax import lax
from jax.experimental import pallas as pl
from jax.experimental.pallas import tpu as pltpu
from jax.experimental.pallas import tpu_sc as plsc

_B = 16384
_D = 32
_INFO = plsc.get_sparse_core_info()
_NC = _INFO.num_cores
_NS = _INFO.num_subcores
_L = _INFO.num_lanes
_NW = _NC * _NS
_BPW = _B // _NW
_C = _L
_NCHUNK = _BPW // _C


def _rsqrt(x):
    i = plsc.bitcast(x, jnp.int32)
    i = 0x5F3759DF - lax.shift_right_logical(i, 1)
    y = plsc.bitcast(i, jnp.float32)
    for _ in range(3):
        y = y * (1.5 - 0.5 * x * y * y)
    return y


def _renorm_scale(sumsq):
    r = _rsqrt(sumsq)
    norm = sumsq * r
    inv = 1.0 / (norm + 1e-7)
    return jnp.where(norm > 1.0, inv, jnp.ones_like(norm))


_MESH = plsc.VectorSubcoreMesh(core_axis_name="c", subcore_axis_name="s")


@functools.partial(
    pl.kernel,
    mesh=_MESH,
    compiler_params=pltpu.CompilerParams(
        needs_layout_passes=False, use_tc_tiling_on_sc=False),
    out_type=jax.ShapeDtypeStruct((_B,), jnp.float32),
    scratch_types=[
        pltpu.VMEM((_BPW,), jnp.int32),
        pltpu.VMEM((_BPW,), jnp.int32),
        pltpu.VMEM((_D, _L), jnp.float32),
        pltpu.VMEM((_D, _L), jnp.float32),
        pltpu.VMEM((_D, _L), jnp.float32),
        pltpu.VMEM((_D, _L), jnp.float32),
        pltpu.VMEM((_BPW,), jnp.float32),
        pltpu.SemaphoreType.DMA,
        pltpu.SemaphoreType.DMA,
    ],
)
def _mf_kernel(users_hbm, items_hbm, utabt_hbm, itabt_hbm, out_hbm,
               uidx_s, iidx_s,
               au_v, av_v, bu_v, bv_v, out_v, sem_a, sem_b):
    wid = lax.axis_index("s") * _NC + lax.axis_index("c")
    base = wid * _BPW

    pltpu.sync_copy(users_hbm.at[pl.ds(base, _BPW)], uidx_s)
    pltpu.sync_copy(items_hbm.at[pl.ds(base, _BPW)], iidx_s)

    def fire(c, bufu, bufv, sem):
        sl = pl.ds(c * _C, _C)
        for f in range(_D):
            pltpu.async_copy(utabt_hbm.at[f].at[uidx_s.at[sl]],
                             bufu.at[f], sem)
            pltpu.async_copy(itabt_hbm.at[f].at[iidx_s.at[sl]],
                             bufv.at[f], sem)

    def wait_pair(bufu, bufv, sem):
        for f in range(_D):
            pltpu.make_async_copy(utabt_hbm.at[f].at[pl.ds(0, _C)],
                                  bufu.at[f], sem).wait()
            pltpu.make_async_copy(itabt_hbm.at[f].at[pl.ds(0, _C)],
                                  bufv.at[f], sem).wait()

    def compute(c, bufu, bufv):
        uu = jnp.zeros((_L,), jnp.float32)
        vv = jnp.zeros((_L,), jnp.float32)
        uv = jnp.zeros((_L,), jnp.float32)
        for f in range(_D):
            u = bufu[f]
            v = bufv[f]
            uu = uu + u * u
            vv = vv + v * v
            uv = uv + u * v
        scale = _renorm_scale(uu) * _renorm_scale(vv)
        out_v[pl.ds(c * _C, _L)] = uv * scale

    fire(0, au_v, av_v, sem_a)

    def body(i, carry):
        c0 = 2 * i
        fire(c0 + 1, bu_v, bv_v, sem_b)
        wait_pair(au_v, av_v, sem_a)
        compute(c0, au_v, av_v)

        @pl.when(i < _NCHUNK // 2 - 1)
        def _():
            fire(c0 + 2, au_v, av_v, sem_a)

        wait_pair(bu_v, bv_v, sem_b)
        compute(c0 + 1, bu_v, bv_v)
        return carry

    lax.fori_loop(0, _NCHUNK // 2, body, 0)

    pltpu.sync_copy(out_v, out_hbm.at[pl.ds(base, _BPW)])


def kernel(users, items, user_table, item_table):
    return _mf_kernel(users.astype(jnp.int32), items.astype(jnp.int32),
                      user_table.T, item_table.T)


# speedup vs baseline: 2.9018x; 2.9018x over previous
import functools

import jax
import jax.numpy as jnp
from jax import lax
from jax.experimental import pallas as pl
from jax.experimental.pallas import tpu as pltpu
from jax.experimental.pallas import tpu_sc as plsc

_B = 16384
_D = 32
_ROWS = 1000000
_LB = 128
_NBLK = _ROWS // _LB
_TAIL = _ROWS - _NBLK * _LB
_INFO = plsc.get_sparse_core_info()
_NC = _INFO.num_cores
_NS = _INFO.num_subcores
_L = _INFO.num_lanes
_NW = _NC * _NS
_BPW = _B // _NW
_CHUNK = 128
_NCH = _BPW // _CHUNK
_GROUPS = _BPW // _L
_BASEBLK = _NBLK // _NW
_EXTRA = _NBLK - _BASEBLK * _NW

_MESH = plsc.VectorSubcoreMesh(core_axis_name="c", subcore_axis_name="s")


def _rsqrt(x):
    i = plsc.bitcast(x, jnp.int32)
    i = 0x5F3759DF - lax.shift_right_logical(i, 1)
    y = plsc.bitcast(i, jnp.float32)
    for _ in range(3):
        y = y * (1.5 - 0.5 * x * y * y)
    return y


def _renorm_scale(sumsq):
    r = _rsqrt(sumsq)
    norm = sumsq * r
    inv = 1.0 / (norm + 1e-7)
    return jnp.where(norm > 1.0, inv, jnp.ones_like(norm))


@functools.partial(
    pl.kernel,
    mesh=_MESH,
    compiler_params=pltpu.CompilerParams(
        needs_layout_passes=False, use_tc_tiling_on_sc=True),
    out_type=(jax.ShapeDtypeStruct((_ROWS * _D,), jnp.float32),
              jax.ShapeDtypeStruct((_ROWS * _D,), jnp.float32)),
    scratch_types=[
        pltpu.VMEM((_D, _LB), jnp.float32),
        pltpu.VMEM((_LB * _D,), jnp.float32),
    ],
)
def _transpose_kernel(utabt_hbm, itabt_hbm, tailu_hbm, tailv_hbm,
                      flatu_hbm, flatv_hbm, in_v, out_v):
    wid = lax.axis_index("s") * _NC + lax.axis_index("c")
    nblk = _BASEBLK + jnp.where(wid < _EXTRA, 1, 0)
    lane32 = lax.iota(jnp.int32, _L) * _D

    def table(tabt_hbm, flat_hbm, tail_hbm):
        def block_body(b, carry):
            k = b * _NW + wid
            col0 = pl.multiple_of(k * _LB, _LB)
            pltpu.sync_copy(tabt_hbm.at[:, pl.ds(col0, _LB)], in_v)
            for f in range(_D):
                for c in range(_LB // _L):
                    v = in_v[f, pl.ds(c * _L, _L)]
                    idx = lane32 + (c * _L * _D + f)
                    plsc.store_scatter(out_v, [idx], v)
            pltpu.sync_copy(out_v, flat_hbm.at[pl.ds(k * _LB * _D,
                                                     _LB * _D)])
            return carry

        lax.fori_loop(0, nblk, block_body, 0)

        @pl.when(wid == _NW - 1)
        def _():
            pltpu.sync_copy(tail_hbm, out_v.at[pl.ds(0, _TAIL * _D)])
            pltpu.sync_copy(out_v.at[pl.ds(0, _TAIL * _D)],
                            flat_hbm.at[pl.ds(_NBLK * _LB * _D,
                                              _TAIL * _D)])

    table(utabt_hbm, flatu_hbm, tailu_hbm)
    table(itabt_hbm, flatv_hbm, tailv_hbm)


@functools.partial(
    pl.kernel,
    mesh=_MESH,
    compiler_params=pltpu.CompilerParams(
        needs_layout_passes=False, use_tc_tiling_on_sc=False),
    out_type=jax.ShapeDtypeStruct((_B,), jnp.float32),
    scratch_types=[
        pltpu.VMEM((_BPW,), jnp.int32),
        pltpu.VMEM((_BPW,), jnp.int32),
        pltpu.VMEM((_BPW, _D), jnp.float32),
        pltpu.VMEM((_BPW, _D), jnp.float32),
        pltpu.VMEM((_BPW,), jnp.float32),
        pltpu.SemaphoreType.DMA,
    ],
)
def _lookup_kernel(users_hbm, items_hbm, utab_hbm, itab_hbm, out_hbm,
                   uidx_v, iidx_v, urows_v, vrows_v, out_v, sem):
    wid = lax.axis_index("s") * _NC + lax.axis_index("c")
    base = wid * _BPW

    pltpu.sync_copy(users_hbm.at[pl.ds(base, _BPW)], uidx_v)
    pltpu.sync_copy(items_hbm.at[pl.ds(base, _BPW)], iidx_v)

    copies = []
    for c in range(_NCH):
        sl = pl.ds(c * _CHUNK, _CHUNK)
        copies.append(
            pltpu.async_copy(utab_hbm.at[uidx_v.at[sl]], urows_v.at[sl], sem))
        copies.append(
            pltpu.async_copy(itab_hbm.at[iidx_v.at[sl]], vrows_v.at[sl], sem))
    for cp in copies:
        cp.wait()

    lane = lax.iota(jnp.int32, _L)

    def group_body(g, carry):
        row0 = g * _L
        uu = jnp.zeros((_L,), jnp.float32)
        vv = jnp.zeros((_L,), jnp.float32)
        uv = jnp.zeros((_L,), jnp.float32)
        for e in range(_L):
            r = row0 + e
            u_lo = urows_v[r, pl.ds(0, _L)]
            u_hi = urows_v[r, pl.ds(_L, _L)]
            v_lo = vrows_v[r, pl.ds(0, _L)]
            v_hi = vrows_v[r, pl.ds(_L, _L)]
            p_uu = u_lo * u_lo + u_hi * u_hi
            p_vv = v_lo * v_lo + v_hi * v_hi
            p_uv = u_lo * v_lo + u_hi * v_hi
            m = lane == e
            uu = jnp.where(m, jnp.sum(p_uu), uu)
            vv = jnp.where(m, jnp.sum(p_vv), vv)
            uv = jnp.where(m, jnp.sum(p_uv), uv)
        su = _renorm_scale(uu)
        sv = _renorm_scale(vv)
        out_v[pl.ds(row0, _L)] = uv * su * sv
        return carry

    lax.fori_loop(0, _GROUPS, group_body, 0)

    pltpu.sync_copy(out_v, out_hbm.at[pl.ds(base, _BPW)])


def kernel(users, items, user_table, item_table):
    tailu = user_table[_NBLK * _LB:].reshape(_TAIL * _D)
    tailv = item_table[_NBLK * _LB:].reshape(_TAIL * _D)
    flat_u, flat_v = _transpose_kernel(user_table.T, item_table.T,
                                       tailu, tailv)
    return _lookup_kernel(users.astype(jnp.int32), items.astype(jnp.int32),
                          flat_u.reshape(_ROWS, _D),
                          flat_v.reshape(_ROWS, _D))
